# baseline (speedup 1.0000x reference)
import math

import jax
import jax.numpy as jnp
from jax import lax
from jax.experimental import pallas as pl
from jax.experimental.pallas import tpu as pltpu
from jax.experimental.pallas import tpu_sc as plsc

_DIM = 512
_SCALE = math.sqrt(_DIM)
_NC, _NS, _L = 2, 16, 16
_NW = _NC * _NS
_CHUNK = 40


def _make_gather(B):
    b_per_w = B // _NW
    n_chunks = b_per_w // _CHUNK
    mesh = plsc.VectorSubcoreMesh(
        core_axis_name="c", subcore_axis_name="s",
        num_cores=_NC, num_subcores=_NS)

    def body(idx_hbm, table_hbm, out_hbm, idx_v,
             in0, in1, ob0, ob1, si0, si1, so0, so1):
        in_v = (in0, in1)
        out_v = (ob0, ob1)
        s_in = (si0, si1)
        s_out = (so0, so1)
        wid = lax.axis_index("s") * _NC + lax.axis_index("c")
        base = pl.multiple_of(wid * b_per_w, 8)
        pltpu.sync_copy(idx_hbm.at[pl.ds(base, b_per_w)], idx_v)

        def gather_start(c, b):
            off = pl.multiple_of(c * _CHUNK, 8)
            pltpu.async_copy(
                table_hbm.at[idx_v.at[pl.ds(off, _CHUNK)]], in_v[b], s_in[b])

        gather_start(0, 0)
        gather_start(1, 1)

        def pair_body(t, carry):
            for b in range(2):
                c = t * 2 + b
                pltpu.make_async_copy(
                    table_hbm.at[idx_v.at[pl.ds(0, _CHUNK)]],
                    in_v[b], s_in[b]).wait()
                @pl.when(t > 0)
                def _():
                    pltpu.make_async_copy(
                        out_v[b], out_hbm.at[pl.ds(0, _CHUNK)],
                        s_out[b]).wait()

                def row_body(i, cc):
                    for j in range(_DIM // _L):
                        sl = pl.ds(j * _L, _L)
                        out_v[b][i, sl] = in_v[b][i, sl] * _SCALE
                    return cc

                lax.fori_loop(0, _CHUNK, row_body, 0)
                off = pl.multiple_of(base + c * _CHUNK, 8)
                pltpu.async_copy(
                    out_v[b], out_hbm.at[pl.ds(off, _CHUNK)], s_out[b])

                @pl.when(c + 2 < n_chunks)
                def _():
                    gather_start(c + 2, b)
            return carry

        lax.fori_loop(0, n_chunks // 2, pair_body, 0)
        for b in range(2):
            pltpu.make_async_copy(
                out_v[b], out_hbm.at[pl.ds(0, _CHUNK)], s_out[b]).wait()

    return pl.kernel(
        body,
        out_type=jax.ShapeDtypeStruct((B, _DIM), jnp.float32),
        mesh=mesh,
        scratch_types=[
            pltpu.VMEM((b_per_w,), jnp.int32),
            pltpu.VMEM((_CHUNK, _DIM), jnp.float32),
            pltpu.VMEM((_CHUNK, _DIM), jnp.float32),
            pltpu.VMEM((_CHUNK, _DIM), jnp.float32),
            pltpu.VMEM((_CHUNK, _DIM), jnp.float32),
            pltpu.SemaphoreType.DMA,
            pltpu.SemaphoreType.DMA,
            pltpu.SemaphoreType.DMA,
            pltpu.SemaphoreType.DMA,
        ],
    )


_SEQ_BLOCK = 32


def _tc_fold(y, n_seq, seq_len):
    def body(i_ref, o_ref):
        o_ref[...] = i_ref[...].reshape(_SEQ_BLOCK, seq_len, _DIM)

    return pl.pallas_call(
        body,
        grid=(n_seq // _SEQ_BLOCK,),
        in_specs=[pl.BlockSpec((_SEQ_BLOCK * seq_len, _DIM),
                               lambda i: (i, 0))],
        out_specs=pl.BlockSpec((_SEQ_BLOCK, seq_len, _DIM),
                               lambda i: (i, 0, 0)),
        out_shape=jax.ShapeDtypeStruct((n_seq, seq_len, _DIM), jnp.float32),
    )(y)


def kernel(x, table):
    n_seq, seq_len = x.shape
    B = n_seq * seq_len
    idx = x.reshape(B)
    y = _make_gather(B)(idx, table)
    return _tc_fold(y, n_seq, seq_len)


# speedup vs baseline: 1.5407x; 1.5407x over previous
import math

import jax
import jax.numpy as jnp
from jax import lax
from jax.experimental import pallas as pl
from jax.experimental.pallas import tpu as pltpu
from jax.experimental.pallas import tpu_sc as plsc

_DIM = 512
_SCALE = math.sqrt(_DIM)
_NC, _NS, _L = 2, 16, 16
_NW = _NC * _NS
_SBLK = 64


def _make_emb(n_seq, seq_len):
    B = n_seq * seq_len
    b_per_w = B // _NW
    seq_per_w = n_seq // _NW
    n_sb = seq_per_w // _SBLK
    n_chunks = seq_len * n_sb
    mesh = plsc.VectorSubcoreMesh(
        core_axis_name="c", subcore_axis_name="s",
        num_cores=_NC, num_subcores=_NS)

    def body(idx_hbm, table_hbm, out_hbm, idx_v,
             buf0, buf1, si0, si1, so0, so1):
        buf = (buf0, buf1)
        s_in = (si0, si1)
        s_out = (so0, so1)
        wid = lax.axis_index("s") * _NC + lax.axis_index("c")
        base = pl.multiple_of(wid * b_per_w, 8)
        seq0 = wid * seq_per_w
        pltpu.sync_copy(idx_hbm.at[pl.ds(base, b_per_w)], idx_v)

        def gather_start(g, b):
            off = pl.multiple_of(g * _SBLK, 8)
            pltpu.async_copy(
                table_hbm.at[idx_v.at[pl.ds(off, _SBLK)]], buf[b], s_in[b])

        def out_dst(g):
            t = g // n_sb
            sb = g % n_sb
            return out_hbm.at[pl.ds(seq0 + sb * _SBLK, _SBLK), t, :]

        gather_start(0, 0)
        gather_start(1, 1)

        def pair_body(p, carry):
            for b in range(2):
                g = p * 2 + b
                pltpu.make_async_copy(
                    table_hbm.at[idx_v.at[pl.ds(0, _SBLK)]],
                    buf[b], s_in[b]).wait()

                def row_body(i, cc):
                    for j in range(_DIM // _L):
                        sl = pl.ds(j * _L, _L)
                        buf[b][i, sl] = buf[b][i, sl] * _SCALE
                    return cc

                lax.fori_loop(0, _SBLK, row_body, 0)
                pltpu.async_copy(buf[b], out_dst(g), s_out[b])

                @pl.when(g + 2 < n_chunks)
                def _():
                    pltpu.make_async_copy(
                        buf[b], out_dst(0), s_out[b]).wait()
                    gather_start(g + 2, b)
            return carry

        lax.fori_loop(0, n_chunks // 2, pair_body, 0)
        for b in range(2):
            pltpu.make_async_copy(buf[b], out_dst(0), s_out[b]).wait()

    return pl.kernel(
        body,
        out_type=jax.ShapeDtypeStruct((n_seq, seq_len, _DIM), jnp.float32),
        mesh=mesh,
        scratch_types=[
            pltpu.VMEM((b_per_w,), jnp.int32),
            pltpu.VMEM((_SBLK, _DIM), jnp.float32),
            pltpu.VMEM((_SBLK, _DIM), jnp.float32),
            pltpu.SemaphoreType.DMA,
            pltpu.SemaphoreType.DMA,
            pltpu.SemaphoreType.DMA,
            pltpu.SemaphoreType.DMA,
        ],
    )


def kernel(x, table):
    n_seq, seq_len = x.shape
    seq_per_w = n_seq // _NW
    idx = (x.reshape(_NW, seq_per_w, seq_len)
            .transpose(0, 2, 1)
            .reshape(n_seq * seq_len))
    return _make_emb(n_seq, seq_len)(idx, table)


# speedup vs baseline: 1.5451x; 1.0028x over previous
import math

import jax
import jax.numpy as jnp
from jax import lax
from jax.experimental import pallas as pl
from jax.experimental.pallas import tpu as pltpu
from jax.experimental.pallas import tpu_sc as plsc

_DIM = 512
_SCALE = math.sqrt(_DIM)
_NC, _NS, _L = 2, 16, 16
_NW = _NC * _NS
_SBLK = 64


def _make_emb(n_seq, seq_len):
    B = n_seq * seq_len
    b_per_w = B // _NW
    seq_per_w = n_seq // _NW
    n_sb = seq_per_w // _SBLK
    n_chunks = seq_len * n_sb
    mesh = plsc.VectorSubcoreMesh(
        core_axis_name="c", subcore_axis_name="s",
        num_cores=_NC, num_subcores=_NS)

    def body(idx_hbm, table_hbm, out_hbm, idx_v,
             buf0, buf1, si0, si1, so0, so1):
        buf = (buf0, buf1)
        s_in = (si0, si1)
        s_out = (so0, so1)
        wid = lax.axis_index("c") * _NS + lax.axis_index("s")
        base = pl.multiple_of(wid * b_per_w, 8)
        seq0 = wid * seq_per_w
        pltpu.sync_copy(idx_hbm.at[pl.ds(base, b_per_w)], idx_v)

        def gather_start(g, b):
            off = pl.multiple_of(g * _SBLK, 8)
            pltpu.async_copy(
                table_hbm.at[idx_v.at[pl.ds(off, _SBLK)]], buf[b], s_in[b])

        def out_dst(g):
            t = g // n_sb
            sb = g % n_sb
            return out_hbm.at[pl.ds(seq0 + sb * _SBLK, _SBLK), t, :]

        gather_start(0, 0)
        gather_start(1, 1)

        def pair_body(p, carry):
            for b in range(2):
                g = p * 2 + b
                pltpu.make_async_copy(
                    table_hbm.at[idx_v.at[pl.ds(0, _SBLK)]],
                    buf[b], s_in[b]).wait()

                def row_body(i, cc):
                    for j in range(_DIM // _L):
                        sl = pl.ds(j * _L, _L)
                        buf[b][i, sl] = buf[b][i, sl] * _SCALE
                    return cc

                lax.fori_loop(0, _SBLK, row_body, 0)
                pltpu.async_copy(buf[b], out_dst(g), s_out[b])

                @pl.when(g + 2 < n_chunks)
                def _():
                    pltpu.make_async_copy(
                        buf[b], out_dst(0), s_out[b]).wait()
                    gather_start(g + 2, b)
            return carry

        lax.fori_loop(0, n_chunks // 2, pair_body, 0)
        for b in range(2):
            pltpu.make_async_copy(buf[b], out_dst(0), s_out[b]).wait()

    return pl.kernel(
        body,
        out_type=jax.ShapeDtypeStruct((n_seq, seq_len, _DIM), jnp.float32),
        mesh=mesh,
        scratch_types=[
            pltpu.VMEM((b_per_w,), jnp.int32),
            pltpu.VMEM((_SBLK, _DIM), jnp.float32),
            pltpu.VMEM((_SBLK, _DIM), jnp.float32),
            pltpu.SemaphoreType.DMA,
            pltpu.SemaphoreType.DMA,
            pltpu.SemaphoreType.DMA,
            pltpu.SemaphoreType.DMA,
        ],
    )


def kernel(x, table):
    n_seq, seq_len = x.shape
    seq_per_w = n_seq // _NW
    idx = (x.reshape(_NW, seq_per_w, seq_len)
            .transpose(0, 2, 1)
            .reshape(n_seq * seq_len))
    return _make_emb(n_seq, seq_len)(idx, table)
